# baseline (speedup 1.0000x reference)
import jax
import jax.numpy as jnp
from jax import lax
from jax.experimental import pallas as pl

G = 64
DN = ('NDHWC', 'DHWIO', 'NDHWC')


def _conv3(x, W, stride=1):
    return lax.conv_general_dilated(x, W, (stride,) * 3, [(1, 1)] * 3,
                                    dimension_numbers=DN)


def _deconv3(x, W):
    return lax.conv_transpose(x, W, (2, 2, 2), 'SAME', dimension_numbers=DN)


def _pool_mask(m):
    return lax.reduce_window(m, 0.0, lax.max, (1, 3, 3, 3, 1), (1, 2, 2, 2, 1),
                             [(0, 0), (1, 1), (1, 1), (1, 1), (0, 0)])


def _bn_relu(x, mask, g, b):
    cnt = jnp.maximum(jnp.sum(mask), 1.0)
    mean = jnp.sum(x * mask, axis=(0, 1, 2, 3)) / cnt
    xc = (x - mean) * mask
    var = jnp.sum(xc * xc, axis=(0, 1, 2, 3)) / cnt
    y = (x - mean) * lax.rsqrt(var + 1e-5) * g + b
    return jax.nn.relu(y) * mask


def _block(x, mask, p, name):
    y = _conv3(x, p[name + '_W'], 1)
    return _bn_relu(y, mask, p[name + '_g'], p[name + '_b'])


def _down(x, mask, p, name):
    m2 = _pool_mask(mask)
    y = _conv3(x, p[name + '_down_W'], 2)
    y = _bn_relu(y, m2, p[name + '_down_g'], p[name + '_down_b'])
    y = _block(y, m2, p, name + '_conv')
    return y, m2


def _up(x, mask_t, p, name):
    y = _deconv3(x, p[name + '_W'])
    return _bn_relu(y, mask_t, p[name + '_g'], p[name + '_b'])


def _identity_pallas(x):
    def body(x_ref, o_ref):
        o_ref[...] = x_ref[...]
    n = x.shape[0]
    blk = 2000
    pad = (-n) % blk
    xp = jnp.pad(x, ((0, pad), (0, 0)))
    out = pl.pallas_call(
        body,
        grid=(xp.shape[0] // blk,),
        in_specs=[pl.BlockSpec((blk, x.shape[1]), lambda i: (i, 0))],
        out_specs=pl.BlockSpec((blk, x.shape[1]), lambda i: (i, 0)),
        out_shape=jax.ShapeDtypeStruct(xp.shape, x.dtype))(xp)
    return out[:n]


def kernel(features, coords, params):
    cz, cy, cx = coords[:, 0], coords[:, 1], coords[:, 2]
    x = jnp.zeros((1, G, G, G, 1), jnp.float32).at[0, cz, cy, cx, :].add(features)
    m1 = jnp.zeros((1, G, G, G, 1), jnp.float32).at[0, cz, cy, cx, :].set(1.0)
    e1 = _block(x, m1, params, 'enc1_conv')
    e2, m2 = _down(e1, m1, params, 'enc2')
    e3, m3 = _down(e2, m2, params, 'enc3')
    e4, m4 = _down(e3, m3, params, 'enc4')
    b, m5 = _down(e4, m4, params, 'bott')
    d4 = _up(b, m4, params, 'up4')
    d4 = _block(jnp.concatenate([d4, e4], -1), m4, params, 'dec4')
    d3 = _up(d4, m3, params, 'up3')
    d3 = _block(jnp.concatenate([d3, e3], -1), m3, params, 'dec3')
    d2 = _up(d3, m2, params, 'up2')
    d2 = _block(jnp.concatenate([d2, e2], -1), m2, params, 'dec2')
    d1 = _up(d2, m1, params, 'up1')
    d1 = _block(jnp.concatenate([d1, e1], -1), m1, params, 'dec1')
    out = d1[0, cz, cy, cx, :]
    return _identity_pallas(out)


# speedup vs baseline: 2.6110x; 2.6110x over previous
import functools

import jax
import jax.numpy as jnp
from jax import lax
from jax.experimental import pallas as pl
from jax.experimental.pallas import tpu as pltpu

G = 64
WP1 = 66
HW1 = WP1 * WP1


def _affine_relu_mask(x, sc, bi, m):
    return jnp.maximum(x * sc + bi, 0.0) * m


def _conv1_body(*refs, ns, d, wp, identity):
    i = 0
    xs = []
    for _ in range(ns):
        xs.append(refs[i:i + 3]); i += 3
    ms = refs[i:i + 3]; i += 3
    affs = []
    if not identity:
        for _ in range(ns):
            affs.append(refs[i:i + 2]); i += 2
    w_ref = refs[i]; i += 1
    y_ref, s1_ref, s2_ref, cnt_ref = refs[i:i + 4]

    z = pl.program_id(0)

    acc = None
    for k in range(3):
        valid = jnp.where((z + k - 1 >= 0) & (z + k - 1 <= d - 1), 1.0, 0.0)
        if identity:
            t = xs[0][k][0] * valid
        else:
            mk = jnp.minimum(ms[k][0], 1.0) * valid
            pieces = [
                _affine_relu_mask(xs[s][k][0], affs[s][0][...],
                                  affs[s][1][...], mk)
                for s in range(ns)
            ]
            t = pieces[0] if ns == 1 else jnp.concatenate(pieces, 0)
        for dy in range(3):
            for dx in range(3):
                sft = (1 - dy) * wp + (1 - dx)
                tt = jnp.roll(t, sft, axis=1) if sft else t
                tap = (k * 3 + dy) * 3 + dx
                p = jnp.dot(w_ref[tap], tt,
                            preferred_element_type=jnp.float32)
                acc = p if acc is None else acc + p

    mc = jnp.minimum(ms[1][0], 1.0)
    ym = acc * mc
    p1 = jnp.sum(ym, axis=1, keepdims=True)
    p2 = jnp.sum(ym * acc, axis=1, keepdims=True)
    pc = jnp.full((1, 128), jnp.sum(mc))

    y_ref[0] = acc

    @pl.when(z == 0)
    def _():
        s1_ref[...] = jnp.zeros_like(s1_ref)
        s2_ref[...] = jnp.zeros_like(s2_ref)
        cnt_ref[...] = jnp.zeros_like(cnt_ref)

    s1_ref[...] += p1
    s2_ref[...] += p2
    cnt_ref[...] += pc


def _conv1(xs, affines, m, w, co):
    d, _, hw = xs[0].shape
    wp = int(round(hw ** 0.5))
    identity = affines[0] is None
    ns = len(xs)
    cin = sum(x.shape[1] for x in xs)

    def cl3(off):
        return lambda z: (jnp.clip(z + off, 0, d - 1), 0, 0)

    in_specs = []
    args = []
    for x in xs:
        for off in (-1, 0, 1):
            in_specs.append(pl.BlockSpec((1, x.shape[1], hw), cl3(off)))
            args.append(x)
    for off in (-1, 0, 1):
        in_specs.append(pl.BlockSpec((1, 1, hw), cl3(off)))
        args.append(m)
    if not identity:
        for sc, bi in affines:
            for v in (sc, bi):
                in_specs.append(
                    pl.BlockSpec((v.shape[0], 1), lambda z: (0, 0)))
                args.append(v.reshape(-1, 1))
    in_specs.append(pl.BlockSpec((27, co, cin), lambda z: (0, 0, 0)))
    args.append(w.reshape(27, cin, co).transpose(0, 2, 1))

    out_specs = [
        pl.BlockSpec((1, co, hw), lambda z: (z, 0, 0)),
        pl.BlockSpec((co, 1), lambda z: (0, 0)),
        pl.BlockSpec((co, 1), lambda z: (0, 0)),
        pl.BlockSpec((1, 128), lambda z: (0, 0)),
    ]
    out_shape = [
        jax.ShapeDtypeStruct((d, co, hw), jnp.float32),
        jax.ShapeDtypeStruct((co, 1), jnp.float32),
        jax.ShapeDtypeStruct((co, 1), jnp.float32),
        jax.ShapeDtypeStruct((1, 128), jnp.float32),
    ]
    body = functools.partial(_conv1_body, ns=ns, d=d, wp=wp,
                             identity=identity)
    y, s1, s2, cnt = pl.pallas_call(
        body, grid=(d,), in_specs=in_specs, out_specs=out_specs,
        out_shape=out_shape)(*args)
    return y, s1[:, 0], s2[:, 0], cnt[0, 0]


def _down_body(*refs, d, wpp, gc, ci):
    xs = refs[0:3]
    ms = refs[3:6]
    sc_ref, bi_ref, w_ref = refs[6:9]
    y_ref, mo_ref, s1_ref, s2_ref, cnt_ref = refs[9:14]

    z = pl.program_id(0)
    co = y_ref.shape[1]
    hpc = gc + 2

    acc = None
    mp = None
    for k in range(3):
        valid = jnp.where(2 * z + k - 1 >= 0, 1.0, 0.0)
        mk4 = jnp.minimum(ms[k][0], 1.0) * valid
        tp = [
            _affine_relu_mask(xs[k][0, pi * ci:(pi + 1) * ci],
                              sc_ref[...], bi_ref[...], mk4[pi:pi + 1])
            for pi in range(4)
        ]
        for dy in range(3):
            for dx in range(3):
                pi = (dy % 2) * 2 + (dx % 2)
                sft = -((dy // 2) * wpp + (dx // 2))
                tt = jnp.roll(tp[pi], sft, axis=1) if sft else tp[pi]
                mt = (jnp.roll(mk4[pi:pi + 1], sft, axis=1) if sft
                      else mk4[pi:pi + 1])
                tap = (k * 3 + dy) * 3 + dx
                p = jnp.dot(w_ref[tap], tt,
                            preferred_element_type=jnp.float32)
                acc = p if acc is None else acc + p
                mp = mt if mp is None else jnp.maximum(mp, mt)

    idx = lax.broadcasted_iota(jnp.int32, (1, wpp * wpp), 1)
    keep = jnp.where((idx // wpp < gc) & (idx % wpp < gc), 1.0, 0.0)
    mpz = mp * keep

    ym = acc * mpz
    p1 = jnp.sum(ym, axis=1, keepdims=True)
    p2 = jnp.sum(ym * acc, axis=1, keepdims=True)
    pc = jnp.full((1, 128), jnp.sum(mpz))

    y_ref[0] = acc
    mo_ref[0] = mpz

    @pl.when(z == 0)
    def _():
        s1_ref[...] = jnp.zeros_like(s1_ref)
        s2_ref[...] = jnp.zeros_like(s2_ref)
        cnt_ref[...] = jnp.zeros_like(cnt_ref)

    s1_ref[...] += p1
    s2_ref[...] += p2
    cnt_ref[...] += pc


def _to_parity(x, wp):
    d, c, _ = x.shape
    h = wp // 2
    x6 = x.reshape(d, c, h, 2, h, 2)
    return x6.transpose(0, 3, 5, 1, 2, 4).reshape(d, 4 * c, h * h)


def _down(x, affine, m, w, co):
    d, ci, hw = x.shape
    wp = int(round(hw ** 0.5))
    dc, gc = d // 2, (wp - 2) // 2
    wpp = wp // 2
    hwp = wpp * wpp
    hpc = gc + 2
    hwc = hpc * hpc
    sc, bi = affine

    xp = _to_parity(x, wp)
    mp = _to_parity(m, wp)

    def fm3(k):
        return lambda z: (jnp.clip(2 * z + k - 1, 0, d - 1), 0, 0)

    in_specs = []
    args = []
    for k in range(3):
        in_specs.append(pl.BlockSpec((1, 4 * ci, hwp), fm3(k)))
        args.append(xp)
    for k in range(3):
        in_specs.append(pl.BlockSpec((1, 4, hwp), fm3(k)))
        args.append(mp)
    in_specs.append(pl.BlockSpec((ci, 1), lambda z: (0, 0)))
    args.append(sc.reshape(-1, 1))
    in_specs.append(pl.BlockSpec((ci, 1), lambda z: (0, 0)))
    args.append(bi.reshape(-1, 1))
    in_specs.append(pl.BlockSpec((27, co, ci), lambda z: (0, 0, 0)))
    args.append(w.reshape(27, ci, co).transpose(0, 2, 1))

    out_specs = [
        pl.BlockSpec((1, co, hwp), lambda z: (z, 0, 0)),
        pl.BlockSpec((1, 1, hwp), lambda z: (z, 0, 0)),
        pl.BlockSpec((co, 1), lambda z: (0, 0)),
        pl.BlockSpec((co, 1), lambda z: (0, 0)),
        pl.BlockSpec((1, 128), lambda z: (0, 0)),
    ]
    out_shape = [
        jax.ShapeDtypeStruct((dc, co, hwp), jnp.float32),
        jax.ShapeDtypeStruct((dc, 1, hwp), jnp.float32),
        jax.ShapeDtypeStruct((co, 1), jnp.float32),
        jax.ShapeDtypeStruct((co, 1), jnp.float32),
        jax.ShapeDtypeStruct((1, 128), jnp.float32),
    ]
    body = functools.partial(_down_body, d=d, wpp=wpp, gc=gc, ci=ci)
    y, mo, s1, s2, cnt = pl.pallas_call(
        body, grid=(dc,), in_specs=in_specs, out_specs=out_specs,
        out_shape=out_shape)(*args)
    y3 = y.reshape(dc, co, wpp, wpp)[:, :, :gc, :gc]
    y_std = jnp.pad(y3, ((0, 0), (0, 0), (1, 1), (1, 1))).reshape(
        dc, co, hwc)
    m3 = mo.reshape(dc, 1, wpp, wpp)[:, :, :gc, :gc]
    m_std = jnp.pad(m3, ((0, 0), (0, 0), (1, 1), (1, 1))).reshape(
        dc, 1, hwc)
    return y_std, m_std, s1[:, 0], s2[:, 0], cnt[0, 0]


_UP_TAPS = ([(0, -1), (2, 0)], [(1, 0)])


def _up_body(*refs, hc, ci):
    xm1, x0, mm1, m0, mfp_ref, sc_ref, bi_ref, w_ref, \
        y_ref, s1_ref, s2_ref, cnt_ref = refs
    z = pl.program_id(0)
    co = y_ref.shape[1] // 4
    wpc = hc + 2

    def t_of(x_ref, m_ref, valid):
        mk = jnp.minimum(m_ref[0], 1.0) * valid
        return _affine_relu_mask(x_ref[0], sc_ref[...], bi_ref[...], mk)

    tm1 = t_of(xm1, mm1, jnp.where(z >= 1, 1.0, 0.0))
    t0 = t_of(x0, m0, 1.0)
    tsrc = {-1: tm1, 0: t0}

    p1 = jnp.zeros((co, 1), jnp.float32)
    p2 = jnp.zeros((co, 1), jnp.float32)
    pc = 0.0
    for pz in (0, 1):
        mf4 = mfp_ref[pz]
        for py in (0, 1):
            for px in (0, 1):
                q = None
                for kz, jz in _UP_TAPS[pz]:
                    for ky, jy in _UP_TAPS[py]:
                        for kx, jx in _UP_TAPS[px]:
                            sft = -(jy * wpc + jx)
                            t = tsrc[jz]
                            tt = jnp.roll(t, sft, axis=1) if sft else t
                            tap = (kz * 3 + ky) * 3 + kx
                            p = jnp.dot(w_ref[tap], tt,
                                        preferred_element_type=jnp.float32)
                            q = p if q is None else q + p
                pq = py * 2 + px
                y_ref[pz, pq * co:(pq + 1) * co] = q

                mc = jnp.minimum(mf4[pq:pq + 1], 1.0)
                ym = q * mc
                p1 += jnp.sum(ym, axis=1, keepdims=True)
                p2 += jnp.sum(ym * q, axis=1, keepdims=True)
                pc += jnp.sum(mc)

    @pl.when(z == 0)
    def _():
        s1_ref[...] = jnp.zeros_like(s1_ref)
        s2_ref[...] = jnp.zeros_like(s2_ref)
        cnt_ref[...] = jnp.zeros_like(cnt_ref)

    s1_ref[...] += p1
    s2_ref[...] += p2
    cnt_ref[...] += jnp.full((1, 128), pc)


def _from_parity(yp, co, gf, wpc):
    d = yp.shape[0]
    h = gf // 2
    y6 = yp.reshape(d, 2, 2, co, wpc, wpc)[:, :, :, :, 1:1 + h, 1:1 + h]
    t = y6.transpose(0, 3, 4, 1, 5, 2).reshape(d, co, gf, gf)
    return jnp.pad(t, ((0, 0), (0, 0), (1, 1), (1, 1))).reshape(
        d, co, (gf + 2) * (gf + 2))


def _up(x, affine, m_coarse, m_fine, w, co):
    dc, ci, hwc = x.shape
    wpc = int(round(hwc ** 0.5))
    hc = wpc - 2
    gf = 2 * hc
    hwq = wpc * wpc
    sc, bi = affine

    wpf = gf + 2
    mfi = m_fine.reshape(2 * dc, 1, wpf, wpf)[:, :, 1:1 + gf, 1:1 + gf]
    mfp4 = _to_parity(mfi.reshape(2 * dc, 1, gf * gf), gf)
    mfp = jnp.pad(mfp4.reshape(2 * dc, 4, hc, hc),
                  ((0, 0), (0, 0), (1, 1), (1, 1))).reshape(
        2 * dc, 4, hwq)

    in_specs = [
        pl.BlockSpec((1, ci, hwc), lambda z: (jnp.clip(z - 1, 0, dc - 1), 0, 0)),
        pl.BlockSpec((1, ci, hwc), lambda z: (z, 0, 0)),
        pl.BlockSpec((1, 1, hwc), lambda z: (jnp.clip(z - 1, 0, dc - 1), 0, 0)),
        pl.BlockSpec((1, 1, hwc), lambda z: (z, 0, 0)),
        pl.BlockSpec((2, 4, hwq), lambda z: (z, 0, 0)),
        pl.BlockSpec((ci, 1), lambda z: (0, 0)),
        pl.BlockSpec((ci, 1), lambda z: (0, 0)),
        pl.BlockSpec((27, co, ci), lambda z: (0, 0, 0)),
    ]
    args = [x, x, m_coarse, m_coarse, mfp, sc.reshape(-1, 1),
            bi.reshape(-1, 1), w.reshape(27, ci, co).transpose(0, 2, 1)]
    out_specs = [
        pl.BlockSpec((2, 4 * co, hwq), lambda z: (z, 0, 0)),
        pl.BlockSpec((co, 1), lambda z: (0, 0)),
        pl.BlockSpec((co, 1), lambda z: (0, 0)),
        pl.BlockSpec((1, 128), lambda z: (0, 0)),
    ]
    out_shape = [
        jax.ShapeDtypeStruct((2 * dc, 4 * co, hwq), jnp.float32),
        jax.ShapeDtypeStruct((co, 1), jnp.float32),
        jax.ShapeDtypeStruct((co, 1), jnp.float32),
        jax.ShapeDtypeStruct((1, 128), jnp.float32),
    ]
    body = functools.partial(_up_body, hc=hc, ci=ci)
    yp, s1, s2, cnt = pl.pallas_call(
        body, grid=(dc,), in_specs=in_specs, out_specs=out_specs,
        out_shape=out_shape)(*args)
    y = _from_parity(yp, co, gf, wpc)
    return y, s1[:, 0], s2[:, 0], cnt[0, 0]


def _bn_aff(s1, s2, cnt, g, b):
    c = jnp.maximum(cnt, 1.0)
    mean = s1 / c
    var = jnp.maximum(s2 / c - mean * mean, 0.0)
    scale = g * lax.rsqrt(var + 1e-5)
    return scale, b - mean * scale


def _scatter_points(features, fidx):
    xf = jnp.zeros((G * HW1,), jnp.float32).at[fidx].add(features[:, 0])
    mf = jnp.zeros((G * HW1,), jnp.float32).at[fidx].add(1.0)
    return xf.reshape(G, 1, HW1), mf.reshape(G, 1, HW1)


def _gather_points(y, fidx, affine):
    sc, bi = affine
    rows = y.transpose(0, 2, 1).reshape(G * HW1, y.shape[1])[fidx]
    return jnp.maximum(rows * sc[None, :] + bi[None, :], 0.0)


def kernel(features, coords, params):
    p = params
    cz = coords[:, 0].astype(jnp.int32)
    cy = coords[:, 1].astype(jnp.int32)
    cx = coords[:, 2].astype(jnp.int32)
    fidx = cz * HW1 + (cy + 1) * WP1 + (cx + 1)

    xg, m1 = _scatter_points(features, fidx)

    def bn(name, s1, s2, cnt):
        return _bn_aff(s1, s2, cnt, p[name + '_g'], p[name + '_b'])

    e1, s1, s2, c1 = _conv1([xg], [None], m1, p['enc1_conv_W'], 16)
    a_e1 = bn('enc1_conv', s1, s2, c1)

    enc = {1: (e1, a_e1, m1)}
    chans = {1: 16, 2: 32, 3: 64, 4: 128, 5: 256}
    names = {2: 'enc2', 3: 'enc3', 4: 'enc4', 5: 'bott'}
    y, aff, m = e1, a_e1, m1
    for lvl in (2, 3, 4, 5):
        nm = names[lvl]
        yd, m2_, s1, s2, cnt = _down(y, aff, m, p[nm + '_down_W'], chans[lvl])
        a_d = bn(nm + '_down', s1, s2, cnt)
        yc, s1, s2, cnt = _conv1([yd], [a_d], m2_, p[nm + '_conv_W'],
                                 chans[lvl])
        a_c = bn(nm + '_conv', s1, s2, cnt)
        y, aff, m = yc, a_c, m2_
        enc[lvl] = (yc, a_c, m2_)

    for lvl in (4, 3, 2, 1):
        nm_up, nm_dec = f'up{lvl}', f'dec{lvl}'
        e_y, e_aff, m_f = enc[lvl]
        yu, s1, s2, cnt = _up(y, aff, m, m_f, p[nm_up + '_W'], chans[lvl])
        a_u = bn(nm_up, s1, s2, cnt)
        yd, s1, s2, cnt = _conv1([yu, e_y], [a_u, e_aff], m_f,
                                 p[nm_dec + '_W'], chans[lvl])
        aff = bn(nm_dec, s1, s2, cnt)
        y, m = yd, m_f

    return _gather_points(y, fidx, aff)


# speedup vs baseline: 2.7837x; 1.0661x over previous
import functools

import jax
import jax.numpy as jnp
from jax import lax
from jax.experimental import pallas as pl
from jax.experimental.pallas import tpu as pltpu

G = 64
WP1 = 66
HW1 = WP1 * WP1


def _affine_relu_mask(x, sc, bi, m):
    return jnp.maximum(x * sc + bi, 0.0) * m


def _conv1_body(*refs, ns, d, wp, identity):
    i = 0
    xs = []
    for _ in range(ns):
        xs.append(refs[i:i + 3]); i += 3
    ms = refs[i:i + 3]; i += 3
    affs = []
    if not identity:
        for _ in range(ns):
            affs.append(refs[i:i + 2]); i += 2
    w_ref = refs[i]; i += 1
    y_ref, s1_ref, s2_ref, cnt_ref = refs[i:i + 4]

    z = pl.program_id(0)

    acc = None
    for k in range(3):
        valid = jnp.where((z + k - 1 >= 0) & (z + k - 1 <= d - 1), 1.0, 0.0)
        if identity:
            t = xs[0][k][0] * valid
        else:
            mk = jnp.minimum(ms[k][0], 1.0) * valid
            pieces = [
                _affine_relu_mask(xs[s][k][0], affs[s][0][...],
                                  affs[s][1][...], mk)
                for s in range(ns)
            ]
            t = pieces[0] if ns == 1 else jnp.concatenate(pieces, 0)
        for dy in range(3):
            for dx in range(3):
                sft = (1 - dy) * wp + (1 - dx)
                tt = jnp.roll(t, sft, axis=1) if sft else t
                tap = (k * 3 + dy) * 3 + dx
                p = jnp.dot(w_ref[tap], tt,
                            preferred_element_type=jnp.float32)
                acc = p if acc is None else acc + p

    mc = jnp.minimum(ms[1][0], 1.0)
    ym = acc * mc
    p1 = jnp.sum(ym, axis=1, keepdims=True)
    p2 = jnp.sum(ym * acc, axis=1, keepdims=True)
    pc = jnp.full((1, 128), jnp.sum(mc))

    y_ref[0] = acc

    @pl.when(z == 0)
    def _():
        s1_ref[...] = jnp.zeros_like(s1_ref)
        s2_ref[...] = jnp.zeros_like(s2_ref)
        cnt_ref[...] = jnp.zeros_like(cnt_ref)

    s1_ref[...] += p1
    s2_ref[...] += p2
    cnt_ref[...] += pc


def _conv1(xs, affines, m, w, co):
    d, _, hw = xs[0].shape
    wp = int(round(hw ** 0.5))
    identity = affines[0] is None
    ns = len(xs)
    cin = sum(x.shape[1] for x in xs)

    def cl3(off):
        return lambda z: (jnp.clip(z + off, 0, d - 1), 0, 0)

    in_specs = []
    args = []
    for x in xs:
        for off in (-1, 0, 1):
            in_specs.append(pl.BlockSpec((1, x.shape[1], hw), cl3(off)))
            args.append(x)
    for off in (-1, 0, 1):
        in_specs.append(pl.BlockSpec((1, 1, hw), cl3(off)))
        args.append(m)
    if not identity:
        for sc, bi in affines:
            for v in (sc, bi):
                in_specs.append(
                    pl.BlockSpec((v.shape[0], 1), lambda z: (0, 0)))
                args.append(v.reshape(-1, 1))
    in_specs.append(pl.BlockSpec((27, co, cin), lambda z: (0, 0, 0)))
    args.append(w.reshape(27, cin, co).transpose(0, 2, 1))

    out_specs = [
        pl.BlockSpec((1, co, hw), lambda z: (z, 0, 0)),
        pl.BlockSpec((co, 1), lambda z: (0, 0)),
        pl.BlockSpec((co, 1), lambda z: (0, 0)),
        pl.BlockSpec((1, 128), lambda z: (0, 0)),
    ]
    out_shape = [
        jax.ShapeDtypeStruct((d, co, hw), jnp.float32),
        jax.ShapeDtypeStruct((co, 1), jnp.float32),
        jax.ShapeDtypeStruct((co, 1), jnp.float32),
        jax.ShapeDtypeStruct((1, 128), jnp.float32),
    ]
    body = functools.partial(_conv1_body, ns=ns, d=d, wp=wp,
                             identity=identity)
    y, s1, s2, cnt = pl.pallas_call(
        body, grid=(d,), in_specs=in_specs, out_specs=out_specs,
        out_shape=out_shape)(*args)
    return y, s1[:, 0], s2[:, 0], cnt[0, 0]


def _down_body(*refs, d, wpp, gc, ci):
    xs = refs[0:3]
    ms = refs[3:6]
    sc_ref, bi_ref, w_ref = refs[6:9]
    y_ref, mo_ref, s1_ref, s2_ref, cnt_ref = refs[9:14]

    z = pl.program_id(0)
    co = y_ref.shape[1]
    hpc = gc + 2

    acc = None
    mp = None
    for k in range(3):
        valid = jnp.where(2 * z + k - 1 >= 0, 1.0, 0.0)
        mk4 = jnp.minimum(ms[k][0], 1.0) * valid
        tp = [
            _affine_relu_mask(xs[k][0, pi * ci:(pi + 1) * ci],
                              sc_ref[...], bi_ref[...], mk4[pi:pi + 1])
            for pi in range(4)
        ]
        for dy in range(3):
            for dx in range(3):
                pi = (dy % 2) * 2 + (dx % 2)
                sft = -((dy // 2) * wpp + (dx // 2))
                tt = jnp.roll(tp[pi], sft, axis=1) if sft else tp[pi]
                mt = (jnp.roll(mk4[pi:pi + 1], sft, axis=1) if sft
                      else mk4[pi:pi + 1])
                tap = (k * 3 + dy) * 3 + dx
                p = jnp.dot(w_ref[tap], tt,
                            preferred_element_type=jnp.float32)
                acc = p if acc is None else acc + p
                mp = mt if mp is None else jnp.maximum(mp, mt)

    idx = lax.broadcasted_iota(jnp.int32, (1, wpp * wpp), 1)
    keep = jnp.where((idx // wpp < gc) & (idx % wpp < gc), 1.0, 0.0)
    mpz = mp * keep

    ym = acc * mpz
    p1 = jnp.sum(ym, axis=1, keepdims=True)
    p2 = jnp.sum(ym * acc, axis=1, keepdims=True)
    pc = jnp.full((1, 128), jnp.sum(mpz))

    y_ref[0] = acc
    mo_ref[0] = mpz

    @pl.when(z == 0)
    def _():
        s1_ref[...] = jnp.zeros_like(s1_ref)
        s2_ref[...] = jnp.zeros_like(s2_ref)
        cnt_ref[...] = jnp.zeros_like(cnt_ref)

    s1_ref[...] += p1
    s2_ref[...] += p2
    cnt_ref[...] += pc


def _to_parity(x, wp):
    d, c, _ = x.shape
    h = wp // 2
    x6 = x.reshape(d, c, h, 2, h, 2)
    return x6.transpose(0, 3, 5, 1, 2, 4).reshape(d, 4 * c, h * h)


def _down(x, affine, m, w, co):
    d, ci, hw = x.shape
    wp = int(round(hw ** 0.5))
    dc, gc = d // 2, (wp - 2) // 2
    wpp = wp // 2
    hwp = wpp * wpp
    hpc = gc + 2
    hwc = hpc * hpc
    sc, bi = affine

    xp = _to_parity(x, wp)
    mp = _to_parity(m, wp)

    def fm3(k):
        return lambda z: (jnp.clip(2 * z + k - 1, 0, d - 1), 0, 0)

    in_specs = []
    args = []
    for k in range(3):
        in_specs.append(pl.BlockSpec((1, 4 * ci, hwp), fm3(k)))
        args.append(xp)
    for k in range(3):
        in_specs.append(pl.BlockSpec((1, 4, hwp), fm3(k)))
        args.append(mp)
    in_specs.append(pl.BlockSpec((ci, 1), lambda z: (0, 0)))
    args.append(sc.reshape(-1, 1))
    in_specs.append(pl.BlockSpec((ci, 1), lambda z: (0, 0)))
    args.append(bi.reshape(-1, 1))
    in_specs.append(pl.BlockSpec((27, co, ci), lambda z: (0, 0, 0)))
    args.append(w.reshape(27, ci, co).transpose(0, 2, 1))

    out_specs = [
        pl.BlockSpec((1, co, hwp), lambda z: (z, 0, 0)),
        pl.BlockSpec((1, 1, hwp), lambda z: (z, 0, 0)),
        pl.BlockSpec((co, 1), lambda z: (0, 0)),
        pl.BlockSpec((co, 1), lambda z: (0, 0)),
        pl.BlockSpec((1, 128), lambda z: (0, 0)),
    ]
    out_shape = [
        jax.ShapeDtypeStruct((dc, co, hwp), jnp.float32),
        jax.ShapeDtypeStruct((dc, 1, hwp), jnp.float32),
        jax.ShapeDtypeStruct((co, 1), jnp.float32),
        jax.ShapeDtypeStruct((co, 1), jnp.float32),
        jax.ShapeDtypeStruct((1, 128), jnp.float32),
    ]
    body = functools.partial(_down_body, d=d, wpp=wpp, gc=gc, ci=ci)
    y, mo, s1, s2, cnt = pl.pallas_call(
        body, grid=(dc,), in_specs=in_specs, out_specs=out_specs,
        out_shape=out_shape)(*args)
    y3 = y.reshape(dc, co, wpp, wpp)[:, :, :gc, :gc]
    y_std = jnp.pad(y3, ((0, 0), (0, 0), (1, 1), (1, 1))).reshape(
        dc, co, hwc)
    m3 = mo.reshape(dc, 1, wpp, wpp)[:, :, :gc, :gc]
    m_std = jnp.pad(m3, ((0, 0), (0, 0), (1, 1), (1, 1))).reshape(
        dc, 1, hwc)
    return y_std, m_std, s1[:, 0], s2[:, 0], cnt[0, 0]


_UP_TAPS = ([(0, -1), (2, 0)], [(1, 0)])


def _up_body(*refs, hc, ci):
    xm1, x0, mm1, m0, mfp_ref, sc_ref, bi_ref, w_ref, \
        y_ref, s1_ref, s2_ref, cnt_ref = refs
    z = pl.program_id(0)
    co = y_ref.shape[1] // 4
    wpc = hc + 2

    def t_of(x_ref, m_ref, valid):
        mk = jnp.minimum(m_ref[0], 1.0) * valid
        return _affine_relu_mask(x_ref[0], sc_ref[...], bi_ref[...], mk)

    tm1 = t_of(xm1, mm1, jnp.where(z >= 1, 1.0, 0.0))
    t0 = t_of(x0, m0, 1.0)
    tsrc = {-1: tm1, 0: t0}

    p1 = jnp.zeros((co, 1), jnp.float32)
    p2 = jnp.zeros((co, 1), jnp.float32)
    pc = 0.0
    for pz in (0, 1):
        mf4 = mfp_ref[pz]
        for py in (0, 1):
            for px in (0, 1):
                q = None
                for kz, jz in _UP_TAPS[pz]:
                    for ky, jy in _UP_TAPS[py]:
                        for kx, jx in _UP_TAPS[px]:
                            sft = -(jy * wpc + jx)
                            t = tsrc[jz]
                            tt = jnp.roll(t, sft, axis=1) if sft else t
                            tap = (kz * 3 + ky) * 3 + kx
                            p = jnp.dot(w_ref[tap], tt,
                                        preferred_element_type=jnp.float32)
                            q = p if q is None else q + p
                pq = py * 2 + px
                y_ref[pz, pq * co:(pq + 1) * co] = q

                mc = jnp.minimum(mf4[pq:pq + 1], 1.0)
                ym = q * mc
                p1 += jnp.sum(ym, axis=1, keepdims=True)
                p2 += jnp.sum(ym * q, axis=1, keepdims=True)
                pc += jnp.sum(mc)

    @pl.when(z == 0)
    def _():
        s1_ref[...] = jnp.zeros_like(s1_ref)
        s2_ref[...] = jnp.zeros_like(s2_ref)
        cnt_ref[...] = jnp.zeros_like(cnt_ref)

    s1_ref[...] += p1
    s2_ref[...] += p2
    cnt_ref[...] += jnp.full((1, 128), pc)


def _from_parity(yp, co, gf, wpc):
    d = yp.shape[0]
    h = gf // 2
    y6 = yp.reshape(d, 2, 2, co, wpc, wpc)[:, :, :, :, 1:1 + h, 1:1 + h]
    t = y6.transpose(0, 3, 4, 1, 5, 2).reshape(d, co, gf, gf)
    return jnp.pad(t, ((0, 0), (0, 0), (1, 1), (1, 1))).reshape(
        d, co, (gf + 2) * (gf + 2))


def _up(x, affine, m_coarse, m_fine, w, co):
    dc, ci, hwc = x.shape
    wpc = int(round(hwc ** 0.5))
    hc = wpc - 2
    gf = 2 * hc
    hwq = wpc * wpc
    sc, bi = affine

    wpf = gf + 2
    mfi = m_fine.reshape(2 * dc, 1, wpf, wpf)[:, :, 1:1 + gf, 1:1 + gf]
    mfp4 = _to_parity(mfi.reshape(2 * dc, 1, gf * gf), gf)
    mfp = jnp.pad(mfp4.reshape(2 * dc, 4, hc, hc),
                  ((0, 0), (0, 0), (1, 1), (1, 1))).reshape(
        2 * dc, 4, hwq)

    in_specs = [
        pl.BlockSpec((1, ci, hwc), lambda z: (jnp.clip(z - 1, 0, dc - 1), 0, 0)),
        pl.BlockSpec((1, ci, hwc), lambda z: (z, 0, 0)),
        pl.BlockSpec((1, 1, hwc), lambda z: (jnp.clip(z - 1, 0, dc - 1), 0, 0)),
        pl.BlockSpec((1, 1, hwc), lambda z: (z, 0, 0)),
        pl.BlockSpec((2, 4, hwq), lambda z: (z, 0, 0)),
        pl.BlockSpec((ci, 1), lambda z: (0, 0)),
        pl.BlockSpec((ci, 1), lambda z: (0, 0)),
        pl.BlockSpec((27, co, ci), lambda z: (0, 0, 0)),
    ]
    args = [x, x, m_coarse, m_coarse, mfp, sc.reshape(-1, 1),
            bi.reshape(-1, 1), w.reshape(27, ci, co).transpose(0, 2, 1)]
    out_specs = [
        pl.BlockSpec((2, 4 * co, hwq), lambda z: (z, 0, 0)),
        pl.BlockSpec((co, 1), lambda z: (0, 0)),
        pl.BlockSpec((co, 1), lambda z: (0, 0)),
        pl.BlockSpec((1, 128), lambda z: (0, 0)),
    ]
    out_shape = [
        jax.ShapeDtypeStruct((2 * dc, 4 * co, hwq), jnp.float32),
        jax.ShapeDtypeStruct((co, 1), jnp.float32),
        jax.ShapeDtypeStruct((co, 1), jnp.float32),
        jax.ShapeDtypeStruct((1, 128), jnp.float32),
    ]
    body = functools.partial(_up_body, hc=hc, ci=ci)
    yp, s1, s2, cnt = pl.pallas_call(
        body, grid=(dc,), in_specs=in_specs, out_specs=out_specs,
        out_shape=out_shape)(*args)
    y = _from_parity(yp, co, gf, wpc)
    return y, s1[:, 0], s2[:, 0], cnt[0, 0]


def _bn_aff(s1, s2, cnt, g, b):
    c = jnp.maximum(cnt, 1.0)
    mean = s1 / c
    var = jnp.maximum(s2 / c - mean * mean, 0.0)
    scale = g * lax.rsqrt(var + 1e-5)
    return scale, b - mean * scale


def _scatter_points(features, fidx):
    pay = jnp.stack([features[:, 0], jnp.ones_like(features[:, 0])], -1)
    sm = jnp.zeros((G * HW1, 2), jnp.float32).at[fidx].add(pay)
    xm = sm.reshape(G, HW1, 2)
    return (xm[:, :, 0].reshape(G, 1, HW1),
            xm[:, :, 1].reshape(G, 1, HW1))


def _gather_points(y, fidx, affine):
    sc, bi = affine
    rows = y.transpose(0, 2, 1).reshape(G * HW1, y.shape[1])[fidx]
    return jnp.maximum(rows * sc[None, :] + bi[None, :], 0.0)


def kernel(features, coords, params):
    p = params
    cz = coords[:, 0].astype(jnp.int32)
    cy = coords[:, 1].astype(jnp.int32)
    cx = coords[:, 2].astype(jnp.int32)
    fidx = cz * HW1 + (cy + 1) * WP1 + (cx + 1)

    xg, m1 = _scatter_points(features, fidx)

    def bn(name, s1, s2, cnt):
        return _bn_aff(s1, s2, cnt, p[name + '_g'], p[name + '_b'])

    e1, s1, s2, c1 = _conv1([xg], [None], m1, p['enc1_conv_W'], 16)
    a_e1 = bn('enc1_conv', s1, s2, c1)

    enc = {1: (e1, a_e1, m1)}
    chans = {1: 16, 2: 32, 3: 64, 4: 128, 5: 256}
    names = {2: 'enc2', 3: 'enc3', 4: 'enc4', 5: 'bott'}
    y, aff, m = e1, a_e1, m1
    for lvl in (2, 3, 4, 5):
        nm = names[lvl]
        yd, m2_, s1, s2, cnt = _down(y, aff, m, p[nm + '_down_W'], chans[lvl])
        a_d = bn(nm + '_down', s1, s2, cnt)
        yc, s1, s2, cnt = _conv1([yd], [a_d], m2_, p[nm + '_conv_W'],
                                 chans[lvl])
        a_c = bn(nm + '_conv', s1, s2, cnt)
        y, aff, m = yc, a_c, m2_
        enc[lvl] = (yc, a_c, m2_)

    for lvl in (4, 3, 2, 1):
        nm_up, nm_dec = f'up{lvl}', f'dec{lvl}'
        e_y, e_aff, m_f = enc[lvl]
        yu, s1, s2, cnt = _up(y, aff, m, m_f, p[nm_up + '_W'], chans[lvl])
        a_u = bn(nm_up, s1, s2, cnt)
        yd, s1, s2, cnt = _conv1([yu, e_y], [a_u, e_aff], m_f,
                                 p[nm_dec + '_W'], chans[lvl])
        aff = bn(nm_dec, s1, s2, cnt)
        y, m = yd, m_f

    return _gather_points(y, fidx, aff)
